# Initial kernel scaffold; baseline (speedup 1.0000x reference)
import jax
import jax.numpy as jnp
from jax.experimental import pallas as pl


def kernel(a, c, conv1_w, conv1_b, conv2_w, conv2_b, fc1_w, fc2_w, fc3_w):
    raise NotImplementedError("write your pallas kernel here")


# speedup vs baseline: 3.0891x; 3.0891x over previous
import numpy as np

import jax
import jax.numpy as jnp
from jax.experimental import pallas as pl
from jax.experimental.pallas import tpu as pltpu

_NEG = 0.01
_BT = 512


def _build_static_indices():
    idx1 = np.full((442, 2176), 156, np.int32)
    ii, jj, oo, di, dj = np.meshgrid(
        np.arange(17), np.arange(17), np.arange(6), np.arange(5), np.arange(5),
        indexing="ij")
    idx1[((ii + di) * 21 + (jj + dj)).ravel(),
         (ii * 128 + jj * 6 + oo).ravel()] = (oo * 25 + di * 5 + dj).ravel()
    for o in range(6):
        idx1[441, np.arange(17 * 17) // 17 * 128 + np.arange(17 * 17) % 17 * 6 + o] = 150 + o
    idx2 = np.full((512, 512), 864, np.int32)
    tt, di2, jj2, oo2, cc, dj2 = np.meshgrid(
        np.arange(2), np.arange(3), np.arange(15), np.arange(16), np.arange(6),
        np.arange(3), indexing="ij")
    r2 = ((tt + di2) * 128 + (jj2 + dj2) * 6 + cc).ravel()
    c2 = (tt * 256 + jj2 * 16 + oo2).ravel()
    v2 = (oo2 * 54 + cc * 9 + di2 * 3 + dj2).ravel()
    idx2[r2, c2] = v2
    idx2l = np.full((384, 256), 864, np.int32)
    t0 = tt.ravel() == 0
    idx2l[r2[t0], c2[t0]] = v2[t0]
    k = np.arange(400, dtype=np.int32)
    o_of_k, s_of_k = k % 16, k // 16
    perm = o_of_k * 25 + s_of_k
    return idx1, idx2, idx2l, perm, o_of_k


_IDX1, _IDX2, _IDX2L, _FC1_PERM, _OIDX = _build_static_indices()


def _leaky(x):
    return jnp.maximum(x, x * _NEG)


def _body(x_ref, m1_ref, m2p_ref, m2l_ref, b2_ref, f1_ref, f2_ref, f3_ref,
          o_ref):
    h1 = jnp.dot(x_ref[...], m1_ref[...], preferred_element_type=jnp.float32)
    h1 = _leaky(h1).astype(jnp.bfloat16)

    vp = [None] * 5

    def fold(i, blk):
        g = i // 3
        vp[g] = blk if vp[g] is None else jnp.maximum(vp[g], blk)

    m2p = m2p_ref[...]
    for p in range(7):
        acc = jnp.dot(h1[:, 256 * p:256 * p + 512], m2p,
                      preferred_element_type=jnp.float32)
        fold(2 * p, acc[:, :256])
        fold(2 * p + 1, acc[:, 256:])
    fold(14, jnp.dot(h1[:, 1792:2176], m2l_ref[...],
                     preferred_element_type=jnp.float32))

    fparts = []
    for g in range(5):
        v = vp[g]
        m = jnp.maximum(jnp.maximum(v[:, 0:208], v[:, 16:224]), v[:, 32:240])
        fparts += [m[:, 0:16], m[:, 48:64], m[:, 96:112], m[:, 144:160],
                   m[:, 192:208]]
    f = jnp.concatenate(fparts, axis=1) + b2_ref[...]
    f = _leaky(f).astype(jnp.bfloat16)

    h = jnp.dot(f, f1_ref[...], preferred_element_type=jnp.float32)
    h = _leaky(h).astype(jnp.bfloat16)
    h = jnp.dot(h, f2_ref[...], preferred_element_type=jnp.float32)
    h = _leaky(h).astype(jnp.bfloat16)
    o_ref[...] = jnp.dot(h, f3_ref[...], preferred_element_type=jnp.float32)


def kernel(a, c, conv1_w, conv1_b, conv2_w, conv2_b, fc1_w, fc2_w, fc3_w):
    B = a.shape[0]
    od = fc3_w.shape[1]
    bt = _BT if B >= _BT else B
    bp = ((B + bt - 1) // bt) * bt

    x = a.reshape(B, 441).astype(jnp.float32)
    x = jnp.concatenate([x, jnp.ones((B, 1), jnp.float32)], axis=1)
    if bp != B:
        x = jnp.pad(x, ((0, bp - B), (0, 0)))
    x = x.astype(jnp.bfloat16)

    src1 = jnp.concatenate([conv1_w.reshape(-1).astype(jnp.float32),
                            conv1_b.astype(jnp.float32),
                            jnp.zeros((1,), jnp.float32)])
    m1 = jnp.take(src1, _IDX1.ravel()).reshape(442, 2176).astype(jnp.bfloat16)
    src2 = jnp.concatenate([conv2_w.reshape(-1).astype(jnp.float32),
                            jnp.zeros((1,), jnp.float32)])
    m2p = jnp.take(src2, _IDX2.ravel()).reshape(512, 512).astype(jnp.bfloat16)
    m2l = jnp.take(src2, _IDX2L.ravel()).reshape(384, 256).astype(jnp.bfloat16)
    b2row = jnp.take(conv2_b.astype(jnp.float32), _OIDX)[None, :]
    f1 = jnp.take(fc1_w.astype(jnp.float32), _FC1_PERM, axis=0
                  ).astype(jnp.bfloat16)
    f2 = fc2_w.astype(jnp.bfloat16)
    f3 = fc3_w.astype(jnp.bfloat16)

    out = pl.pallas_call(
        _body,
        out_shape=jax.ShapeDtypeStruct((bp, od), jnp.float32),
        grid=(bp // bt,),
        in_specs=[
            pl.BlockSpec((bt, 442), lambda i: (i, 0)),
            pl.BlockSpec((442, 2176), lambda i: (0, 0)),
            pl.BlockSpec((512, 512), lambda i: (0, 0)),
            pl.BlockSpec((384, 256), lambda i: (0, 0)),
            pl.BlockSpec((1, 400), lambda i: (0, 0)),
            pl.BlockSpec((400, 128), lambda i: (0, 0)),
            pl.BlockSpec((128, 64), lambda i: (0, 0)),
            pl.BlockSpec((64, od), lambda i: (0, 0)),
        ],
        out_specs=pl.BlockSpec((bt, od), lambda i: (i, 0)),
        compiler_params=pltpu.CompilerParams(
            dimension_semantics=("parallel",)),
    )(x, m1, m2p, m2l, b2row, f1, f2, f3)
    return out[:B]


# speedup vs baseline: 109.0522x; 35.3026x over previous
import numpy as np

import jax
import jax.numpy as jnp
from jax.experimental import pallas as pl
from jax.experimental.pallas import tpu as pltpu

_NEG = 0.01
_BT = 512


def _band(n_out, n_in, n_tap):
    e = np.zeros((n_out, n_in, n_tap), np.float32)
    for b in range(n_in):
        for d in range(n_tap):
            e[b + d, b, d] = 1.0
    return e


_E21 = _band(21, 17, 5)
_E4 = _band(4, 2, 3)
_E17 = _band(17, 15, 3)


def _conv1_operator(conv1_w, conv1_b):
    w1 = conv1_w.reshape(6, 5, 5).astype(jnp.float32)
    tmp = jnp.einsum("rjd,oad->rjoa", _E21, w1)
    m = jnp.einsum("xia,yjoa->xyijo", _E21, tmp)
    m = m.reshape(441, 17, 102)
    m = jnp.pad(m, ((0, 0), (0, 0), (0, 26)))
    bias = jnp.broadcast_to(conv1_b.astype(jnp.float32)[None, None, :],
                            (1, 289, 6)).reshape(1, 17, 17, 6)
    bias = jnp.pad(bias.reshape(1, 17, 102), ((0, 0), (0, 0), (0, 26)))
    return jnp.concatenate([m, bias], axis=0).reshape(442, 2176)


def _conv2_operator(conv2_w):
    w2 = conv2_w.astype(jnp.float32)
    tmp = jnp.einsum("pjd,ocad->pjoca", _E17, w2)
    m = jnp.einsum("rta,pjoca->rpctjo", _E4, tmp)
    m = m.reshape(4, 102, 2, 240)
    m = jnp.pad(m, ((0, 0), (0, 26), (0, 0), (0, 16)))
    return m.reshape(512, 512)


def _leaky(x):
    return jnp.maximum(x, x * _NEG)


def _body(x_ref, m1_ref, m2p_ref, m2l_ref, b2_ref, f1_ref, f2_ref, f3_ref,
          o_ref):
    h1 = jnp.dot(x_ref[...], m1_ref[...], preferred_element_type=jnp.float32)
    h1 = _leaky(h1).astype(jnp.bfloat16)

    vp = [None] * 5

    def fold(i, blk):
        g = i // 3
        vp[g] = blk if vp[g] is None else jnp.maximum(vp[g], blk)

    m2p = m2p_ref[...]
    for p in range(7):
        acc = jnp.dot(h1[:, 256 * p:256 * p + 512], m2p,
                      preferred_element_type=jnp.float32)
        fold(2 * p, acc[:, :256])
        fold(2 * p + 1, acc[:, 256:])
    fold(14, jnp.dot(h1[:, 1792:2176], m2l_ref[...],
                     preferred_element_type=jnp.float32))

    fparts = []
    for g in range(5):
        v = vp[g]
        m = jnp.maximum(jnp.maximum(v[:, 0:208], v[:, 16:224]), v[:, 32:240])
        fparts += [m[:, 0:16], m[:, 48:64], m[:, 96:112], m[:, 144:160],
                   m[:, 192:208]]
    f = jnp.concatenate(fparts, axis=1) + b2_ref[...]
    f = _leaky(f).astype(jnp.bfloat16)

    h = jnp.dot(f, f1_ref[...], preferred_element_type=jnp.float32)
    h = _leaky(h).astype(jnp.bfloat16)
    h = jnp.dot(h, f2_ref[...], preferred_element_type=jnp.float32)
    h = _leaky(h).astype(jnp.bfloat16)
    o_ref[...] = jnp.dot(h, f3_ref[...], preferred_element_type=jnp.float32)


def kernel(a, c, conv1_w, conv1_b, conv2_w, conv2_b, fc1_w, fc2_w, fc3_w):
    B = a.shape[0]
    od = fc3_w.shape[1]
    bt = _BT if B >= _BT else B
    bp = ((B + bt - 1) // bt) * bt

    x = a.reshape(B, 441).astype(jnp.float32)
    x = jnp.concatenate([x, jnp.ones((B, 1), jnp.float32)], axis=1)
    if bp != B:
        x = jnp.pad(x, ((0, bp - B), (0, 0)))
    x = x.astype(jnp.bfloat16)

    m1 = _conv1_operator(conv1_w, conv1_b).astype(jnp.bfloat16)
    m2p = _conv2_operator(conv2_w).astype(jnp.bfloat16)
    m2l = m2p[:384, :256]
    b2row = jnp.broadcast_to(conv2_b.astype(jnp.float32)[None, None, :],
                             (1, 25, 16)).reshape(1, 400)
    f1 = fc1_w.astype(jnp.float32).reshape(16, 25, 128).transpose(1, 0, 2) \
        .reshape(400, 128).astype(jnp.bfloat16)
    f2 = fc2_w.astype(jnp.bfloat16)
    f3 = fc3_w.astype(jnp.bfloat16)

    out = pl.pallas_call(
        _body,
        out_shape=jax.ShapeDtypeStruct((bp, od), jnp.float32),
        grid=(bp // bt,),
        in_specs=[
            pl.BlockSpec((bt, 442), lambda i: (i, 0)),
            pl.BlockSpec((442, 2176), lambda i: (0, 0)),
            pl.BlockSpec((512, 512), lambda i: (0, 0)),
            pl.BlockSpec((384, 256), lambda i: (0, 0)),
            pl.BlockSpec((1, 400), lambda i: (0, 0)),
            pl.BlockSpec((400, 128), lambda i: (0, 0)),
            pl.BlockSpec((128, 64), lambda i: (0, 0)),
            pl.BlockSpec((64, od), lambda i: (0, 0)),
        ],
        out_specs=pl.BlockSpec((bt, od), lambda i: (i, 0)),
        compiler_params=pltpu.CompilerParams(
            dimension_semantics=("parallel",)),
    )(x, m1, m2p, m2l, b2row, f1, f2, f3)
    return out[:B]


# speedup vs baseline: 114.0179x; 1.0455x over previous
import numpy as np

import jax
import jax.numpy as jnp
from jax.experimental import pallas as pl
from jax.experimental.pallas import tpu as pltpu

_NEG = 0.01
_BT = 512


def _band(n_out, n_in, n_tap):
    e = np.zeros((n_out, n_in, n_tap), np.float32)
    for b in range(n_in):
        for d in range(n_tap):
            e[b + d, b, d] = 1.0
    return e


_E21 = _band(21, 17, 5)
_E4 = _band(4, 2, 3)
_E17 = _band(17, 15, 3)


def _conv1_operator(conv1_w, conv1_b):
    w1 = conv1_w.reshape(6, 5, 5).astype(jnp.float32)
    tmp = jnp.einsum("rjd,oad->rjoa", _E21, w1)
    m = jnp.einsum("xia,yjoa->xyijo", _E21, tmp)
    m = m.reshape(441, 17, 102)
    m = jnp.pad(m, ((0, 0), (0, 0), (0, 26)))
    bias = jnp.broadcast_to(conv1_b.astype(jnp.float32)[None, None, :],
                            (1, 289, 6)).reshape(1, 17, 17, 6)
    bias = jnp.pad(bias.reshape(1, 17, 102), ((0, 0), (0, 0), (0, 26)))
    return jnp.concatenate([m, bias], axis=0).reshape(442, 2176)


def _conv2_operator(conv2_w):
    w2 = conv2_w.astype(jnp.float32)
    tmp = jnp.einsum("pjd,ocad->pjoca", _E17, w2)
    m = jnp.einsum("rta,pjoca->rpctjo", _E4, tmp)
    m = m.reshape(4, 102, 2, 240)
    m = jnp.pad(m, ((0, 0), (0, 26), (0, 0), (0, 16)))
    return m.reshape(512, 512)


def _leaky(x):
    return jnp.maximum(x, x * _NEG)


def _body(x_ref, m1_ref, m2p_ref, m2l_ref, b2_ref, f1_ref, f2_ref, f3_ref,
          o_ref):
    x = jnp.pad(x_ref[...].astype(jnp.bfloat16), ((0, 0), (0, 1)),
                constant_values=1)
    h1 = jnp.dot(x, m1_ref[...], preferred_element_type=jnp.float32)
    h1 = _leaky(h1).astype(jnp.bfloat16)

    vp = [None] * 5

    def fold(i, blk):
        g = i // 3
        vp[g] = blk if vp[g] is None else jnp.maximum(vp[g], blk)

    m2p = m2p_ref[...]
    for p in range(7):
        acc = jnp.dot(h1[:, 256 * p:256 * p + 512], m2p,
                      preferred_element_type=jnp.float32)
        fold(2 * p, acc[:, :256])
        fold(2 * p + 1, acc[:, 256:])
    fold(14, jnp.dot(h1[:, 1792:2176], m2l_ref[...],
                     preferred_element_type=jnp.float32))

    fparts = []
    for g in range(5):
        v = vp[g]
        m = jnp.maximum(jnp.maximum(v[:, 0:208], v[:, 16:224]), v[:, 32:240])
        fparts += [m[:, 0:16], m[:, 48:64], m[:, 96:112], m[:, 144:160],
                   m[:, 192:208]]
    f = jnp.concatenate(fparts, axis=1) + b2_ref[...]
    f = _leaky(f).astype(jnp.bfloat16)

    h = jnp.dot(f, f1_ref[...], preferred_element_type=jnp.float32)
    h = _leaky(h).astype(jnp.bfloat16)
    h = jnp.dot(h, f2_ref[...], preferred_element_type=jnp.float32)
    h = _leaky(h).astype(jnp.bfloat16)
    o_ref[...] = jnp.dot(h, f3_ref[...], preferred_element_type=jnp.float32)


def kernel(a, c, conv1_w, conv1_b, conv2_w, conv2_b, fc1_w, fc2_w, fc3_w):
    B = a.shape[0]
    od = fc3_w.shape[1]
    bt = _BT if B >= _BT else B
    bp = ((B + bt - 1) // bt) * bt

    x = a.reshape(B, 441)
    if bp != B:
        x = jnp.pad(x, ((0, bp - B), (0, 0)))

    m1 = _conv1_operator(conv1_w, conv1_b).astype(jnp.bfloat16)
    m2p = _conv2_operator(conv2_w).astype(jnp.bfloat16)
    m2l = m2p[:384, :256]
    b2row = jnp.broadcast_to(conv2_b.astype(jnp.float32)[None, None, :],
                             (1, 25, 16)).reshape(1, 400)
    f1 = fc1_w.astype(jnp.float32).reshape(16, 25, 128).transpose(1, 0, 2) \
        .reshape(400, 128).astype(jnp.bfloat16)
    f2 = fc2_w.astype(jnp.bfloat16)
    f3 = fc3_w.astype(jnp.bfloat16)

    out = pl.pallas_call(
        _body,
        out_shape=jax.ShapeDtypeStruct((bp, od), jnp.float32),
        grid=(bp // bt,),
        in_specs=[
            pl.BlockSpec((bt, 441), lambda i: (i, 0)),
            pl.BlockSpec((442, 2176), lambda i: (0, 0)),
            pl.BlockSpec((512, 512), lambda i: (0, 0)),
            pl.BlockSpec((384, 256), lambda i: (0, 0)),
            pl.BlockSpec((1, 400), lambda i: (0, 0)),
            pl.BlockSpec((400, 128), lambda i: (0, 0)),
            pl.BlockSpec((128, 64), lambda i: (0, 0)),
            pl.BlockSpec((64, od), lambda i: (0, 0)),
        ],
        out_specs=pl.BlockSpec((bt, od), lambda i: (i, 0)),
        compiler_params=pltpu.CompilerParams(
            dimension_semantics=("parallel",)),
    )(x, m1, m2p, m2l, b2row, f1, f2, f3)
    return out[:B]
